# baseline (speedup 1.0000x reference)
import functools

import jax
import jax.numpy as jnp
from jax import lax
from jax.experimental import pallas as pl
from jax.experimental.pallas import tpu as pltpu
from jax.experimental.pallas import tpu_sc as plsc

NUM_POINTS = 1000000
DIMS = 64
BATCH = 16384

NC = 2
NS = 16
NW = NC * NS
BPW = BATCH // NW
CHUNK = 128
NCHUNK = BPW // CHUNK
LANES = 16


def _sc_body(lidx_hbm, ridx_hbm, emb_hbm, blhs_hbm, brhs_hbm, out_hbm,
             lidx_v, ridx_v, lrows_v, rrows_v, lb_v, rb_v, out_v, m_v, sem):
    wid = lax.axis_index("s") * NC + lax.axis_index("c")
    row0 = wid * NCHUNK
    base = wid * BPW

    pltpu.sync_copy(lidx_hbm.at[pl.ds(row0, NCHUNK)], lidx_v)
    pltpu.sync_copy(ridx_hbm.at[pl.ds(row0, NCHUNK)], ridx_v)

    copies = []
    for c in range(NCHUNK):
        copies.append(pltpu.async_copy(
            emb_hbm.at[lidx_v.at[c]], lrows_v.at[pl.ds(c * CHUNK, CHUNK)], sem))
        copies.append(pltpu.async_copy(
            emb_hbm.at[ridx_v.at[c]], rrows_v.at[pl.ds(c * CHUNK, CHUNK)], sem))
        copies.append(pltpu.async_copy(
            blhs_hbm.at[lidx_v.at[c]], lb_v.at[pl.ds(c * CHUNK, CHUNK)], sem))
        copies.append(pltpu.async_copy(
            brhs_hbm.at[ridx_v.at[c]], rb_v.at[pl.ds(c * CHUNK, CHUNK)], sem))
    for cp in copies:
        cp.wait()

    lane = lax.iota(jnp.int32, LANES)

    def block(b, carry):
        for j in range(LANES):
            e = b * LANES + j
            acc = jnp.zeros((LANES,), jnp.float32)
            for k in range(DIMS // LANES):
                lv = lrows_v[e, pl.ds(k * LANES, LANES)]
                rv = rrows_v[e, pl.ds(k * LANES, LANES)]
                d = lv - rv
                acc = acc + d * d
            m_v[pl.ds(j * LANES, LANES)] = acc
        sqv = jnp.zeros((LANES,), jnp.float32)
        for k in range(LANES):
            sqv = sqv + plsc.load_gather(m_v, [lane * LANES + k])
        lb = lb_v[pl.ds(b * LANES, LANES)]
        rb = rb_v[pl.ds(b * LANES, LANES)]
        out_v[pl.ds(b * LANES, LANES)] = lb + rb - sqv
        return carry

    lax.fori_loop(0, BPW // LANES, block, 0)

    pltpu.sync_copy(out_v, out_hbm.at[pl.ds(base, BPW)])


@jax.jit
def _run(lidx, ridx, embeddings, bias_lhs, bias_rhs):
    mesh = plsc.VectorSubcoreMesh(core_axis_name="c", subcore_axis_name="s")
    f = pl.kernel(
        _sc_body,
        out_type=jax.ShapeDtypeStruct((BATCH,), jnp.float32),
        mesh=mesh,
        compiler_params=pltpu.CompilerParams(
            needs_layout_passes=False, use_tc_tiling_on_sc=False),
        scratch_types=[
            pltpu.VMEM((NCHUNK, CHUNK), jnp.int32),
            pltpu.VMEM((NCHUNK, CHUNK), jnp.int32),
            pltpu.VMEM((BPW, DIMS), jnp.float32),
            pltpu.VMEM((BPW, DIMS), jnp.float32),
            pltpu.VMEM((BPW,), jnp.float32),
            pltpu.VMEM((BPW,), jnp.float32),
            pltpu.VMEM((BPW,), jnp.float32),
            pltpu.VMEM((LANES * LANES,), jnp.float32),
            pltpu.SemaphoreType.DMA,
        ],
    )
    return f(lidx, ridx, embeddings, bias_lhs, bias_rhs)


def kernel(input_triplet, embeddings, bias_lhs, bias_rhs):
    lidx = input_triplet[:, 0].reshape(NW * NCHUNK, CHUNK).astype(jnp.int32)
    ridx = input_triplet[:, -1].reshape(NW * NCHUNK, CHUNK).astype(jnp.int32)
    return _run(lidx, ridx, embeddings, bias_lhs, bias_rhs)
